# Initial kernel scaffold; baseline (speedup 1.0000x reference)
import jax
import jax.numpy as jnp
from jax.experimental import pallas as pl


def kernel(ids, kernel):
    raise NotImplementedError("write your pallas kernel here")


# speedup vs baseline: 1.5622x; 1.5622x over previous
import functools

import jax
import jax.numpy as jnp
from jax import lax
from jax.experimental import pallas as pl
from jax.experimental.pallas import tpu as pltpu
from jax.experimental.pallas import tpu_sc as plsc

_DIM = 512


@functools.cache
def _make_lookup(B, D):
    info = plsc.get_sparse_core_info()
    NC, NS = info.num_cores, info.num_subcores
    NW = NC * NS
    assert B % NW == 0
    b_per_w = B // NW
    CB = 80
    assert b_per_w % (2 * CB) == 0
    NP = b_per_w // CB // 2
    mesh = plsc.VectorSubcoreMesh(core_axis_name="c", subcore_axis_name="s")

    @functools.partial(
        pl.kernel,
        mesh=mesh,
        out_type=jax.ShapeDtypeStruct((B, D), jnp.float32),
        scratch_types=[
            pltpu.VMEM((b_per_w,), jnp.int32),
            pltpu.VMEM((CB, D), jnp.float32),
            pltpu.VMEM((CB, D), jnp.float32),
            pltpu.SemaphoreType.DMA,
            pltpu.SemaphoreType.DMA,
        ],
    )
    def lookup(table_hbm, idx_hbm, out_hbm, idx_v, rows0, rows1, sem0, sem1):
        wid = lax.axis_index("s") * NC + lax.axis_index("c")
        base = wid * b_per_w
        pltpu.sync_copy(idx_hbm.at[pl.ds(base, b_per_w)], idx_v)
        pltpu.async_copy(table_hbm.at[idx_v.at[pl.ds(0, CB)]], rows0, sem0)

        def body(p, carry):
            off0 = 2 * p * CB
            off1 = off0 + CB
            pltpu.async_copy(
                table_hbm.at[idx_v.at[pl.ds(off1, CB)]], rows1, sem1)
            pltpu.make_async_copy(
                table_hbm.at[idx_v.at[pl.ds(off0, CB)]], rows0, sem0).wait()
            pltpu.sync_copy(rows0, out_hbm.at[pl.ds(base + off0, CB)])

            @pl.when(p < NP - 1)
            def _():
                pltpu.async_copy(
                    table_hbm.at[idx_v.at[pl.ds(off1 + CB, CB)]], rows0, sem0)

            pltpu.make_async_copy(
                table_hbm.at[idx_v.at[pl.ds(off1, CB)]], rows1, sem1).wait()
            pltpu.sync_copy(rows1, out_hbm.at[pl.ds(base + off1, CB)])
            return carry

        lax.fori_loop(0, NP, body, 0)

    return lookup


def kernel(ids, kernel):
    rows, cols = ids.shape
    B = rows * cols
    idx = ids.reshape(B).astype(jnp.int32)
    out = _make_lookup(B, _DIM)(kernel, idx)
    return out.reshape(rows, cols, _DIM)


# speedup vs baseline: 1.5664x; 1.0027x over previous
import functools

import jax
import jax.numpy as jnp
from jax import lax
from jax.experimental import pallas as pl
from jax.experimental.pallas import tpu as pltpu
from jax.experimental.pallas import tpu_sc as plsc

_DIM = 512
_NB = 4
_CB = 40


@functools.cache
def _make_lookup(B, D):
    info = plsc.get_sparse_core_info()
    NC, NS = info.num_cores, info.num_subcores
    NW = NC * NS
    assert B % NW == 0
    b_per_w = B // NW
    NCH = b_per_w // _CB
    assert b_per_w % (_NB * _CB) == 0
    NP = NCH // _NB
    mesh = plsc.VectorSubcoreMesh(core_axis_name="c", subcore_axis_name="s")

    @functools.partial(
        pl.kernel,
        mesh=mesh,
        out_type=jax.ShapeDtypeStruct((B, D), jnp.float32),
        scratch_types=[
            pltpu.VMEM((b_per_w,), jnp.int32),
            [pltpu.VMEM((_CB, D), jnp.float32) for _ in range(_NB)],
            [pltpu.SemaphoreType.DMA for _ in range(_NB)],
            [pltpu.SemaphoreType.DMA for _ in range(_NB)],
        ],
    )
    def lookup(table_hbm, idx_hbm, out_hbm, idx_v, rows, sg, ss):
        wid = lax.axis_index("s") * NC + lax.axis_index("c")
        base = wid * b_per_w
        pltpu.sync_copy(idx_hbm.at[pl.ds(base, b_per_w)], idx_v)

        def gather(c, j):
            pltpu.async_copy(
                table_hbm.at[idx_v.at[pl.ds(c * _CB, _CB)]], rows[j], sg[j])

        def gather_wait(c, j):
            pltpu.make_async_copy(
                table_hbm.at[idx_v.at[pl.ds(c * _CB, _CB)]], rows[j],
                sg[j]).wait()

        def scatter(c, j):
            pltpu.async_copy(
                rows[j], out_hbm.at[pl.ds(base + c * _CB, _CB)], ss[j])

        def scatter_wait(c, j):
            pltpu.make_async_copy(
                rows[j], out_hbm.at[pl.ds(base + c * _CB, _CB)],
                ss[j]).wait()

        gather(0, 0)
        gather(1, 1)

        def body(p, carry):
            for j in range(_NB):
                c = _NB * p + j
                j2 = (j + 2) % _NB
                @pl.when(c >= 2)
                def _():
                    scatter_wait(c - 2, j2)

                @pl.when(c + 2 < NCH)
                def _():
                    gather(c + 2, j2)

                gather_wait(c, j)
                scatter(c, j)
            return carry

        lax.fori_loop(0, NP, body, 0)
        scatter_wait(NCH - 2, (NCH - 2) % _NB)
        scatter_wait(NCH - 1, (NCH - 1) % _NB)

    return lookup


def kernel(ids, kernel):
    rows, cols = ids.shape
    B = rows * cols
    idx = ids.reshape(B).astype(jnp.int32)
    out = _make_lookup(B, _DIM)(kernel, idx)
    return out.reshape(rows, cols, _DIM)
